# baseline (speedup 1.0000x reference)
import functools

import jax
import jax.numpy as jnp
from jax.experimental import pallas as pl
from jax.experimental.pallas import tpu as pltpu

E = 320000
T = 640000
H = 128
IE = 64
BE_BLK = 2000
BT_BLK = 4000


def _swish(x):
    return x * jax.nn.sigmoid(x)


def _pre_body(x1_ref, rbf0_ref, W_ji_ref, b_ji_ref, W_kj_ref, b_kj_ref,
              W_rbf1_ref, W_rbf2_ref, W_down_ref, x_ji_ref, xd_ref):
    x1 = x1_ref[...]
    x_ji = _swish(jnp.dot(x1, W_ji_ref[...],
                          preferred_element_type=jnp.float32) + b_ji_ref[...])
    x_ji_ref[...] = x_ji
    x_kj = _swish(jnp.dot(x1, W_kj_ref[...],
                          preferred_element_type=jnp.float32) + b_kj_ref[...])
    rbf = jnp.dot(jnp.dot(rbf0_ref[...], W_rbf1_ref[...],
                          preferred_element_type=jnp.float32), W_rbf2_ref[...],
                  preferred_element_type=jnp.float32)
    xd_ref[...] = _swish(jnp.dot(x_kj * rbf, W_down_ref[...],
                                 preferred_element_type=jnp.float32))


def _sbf_body(sbf_ref, W_sbf1_ref, W_sbf2_ref, out_ref):
    out_ref[...] = jnp.dot(
        jnp.dot(sbf_ref[...], W_sbf1_ref[...],
                preferred_element_type=jnp.float32), W_sbf2_ref[...],
        preferred_element_type=jnp.float32)


def _post_body(seg_ref, x_ji_ref, x1_ref, rbf0_ref, W_up_ref,
               rb_W1_ref, rb_b1_ref, rb_W2_ref, rb_b2_ref,
               W_lin_ref, b_lin_ref,
               ra_W1_ref, ra_b1_ref, ra_W2_ref, ra_b2_ref,
               W_rbf_ref, e1_ref, e2_ref):
    xu = _swish(jnp.dot(seg_ref[...], W_up_ref[...],
                        preferred_element_type=jnp.float32))
    e1 = x_ji_ref[...] + xu
    e1 = e1 + _swish(
        jnp.dot(_swish(jnp.dot(e1, rb_W1_ref[0],
                               preferred_element_type=jnp.float32)
                       + rb_b1_ref[...]), rb_W2_ref[0],
                preferred_element_type=jnp.float32) + rb_b2_ref[...])
    e1 = _swish(jnp.dot(e1, W_lin_ref[...],
                        preferred_element_type=jnp.float32)
                + b_lin_ref[...]) + x1_ref[...]
    for li in range(2):
        e1 = e1 + _swish(
            jnp.dot(_swish(jnp.dot(e1, ra_W1_ref[li],
                                   preferred_element_type=jnp.float32)
                           + ra_b1_ref[li:li + 1]), ra_W2_ref[li],
                    preferred_element_type=jnp.float32) + ra_b2_ref[li:li + 1])
    e1_ref[...] = e1
    e2_ref[...] = jnp.dot(rbf0_ref[...], W_rbf_ref[...],
                          preferred_element_type=jnp.float32) * e1


def _full(shape):
    return pl.BlockSpec(shape, lambda i: tuple(0 for _ in shape))


def kernel(x1, x2, rbf0, sbf, idx_kj, idx_ji, W_ji, b_ji, W_kj, b_kj,
           W_rbf1, W_rbf2, W_sbf1, W_sbf2, W_down, W_up, rb_W1, rb_b1,
           rb_W2, rb_b2, W_lin, b_lin, ra_W1, ra_b1, ra_W2, ra_b2, W_rbf):
    nb_e = E // BE_BLK
    nb_t = T // BT_BLK

    x_ji, xd = pl.pallas_call(
        _pre_body,
        grid=(nb_e,),
        in_specs=[
            pl.BlockSpec((BE_BLK, H), lambda i: (i, 0)),
            pl.BlockSpec((BE_BLK, 6), lambda i: (i, 0)),
            _full((H, H)), _full((H,)), _full((H, H)), _full((H,)),
            _full((6, 8)), _full((8, H)), _full((H, IE)),
        ],
        out_specs=[
            pl.BlockSpec((BE_BLK, H), lambda i: (i, 0)),
            pl.BlockSpec((BE_BLK, IE), lambda i: (i, 0)),
        ],
        out_shape=[
            jax.ShapeDtypeStruct((E, H), jnp.float32),
            jax.ShapeDtypeStruct((E, IE), jnp.float32),
        ],
    )(x1, rbf0, W_ji, b_ji, W_kj, b_kj, W_rbf1, W_rbf2, W_down)

    sbf_t = pl.pallas_call(
        _sbf_body,
        grid=(nb_t,),
        in_specs=[
            pl.BlockSpec((BT_BLK, 42), lambda i: (i, 0)),
            _full((42, 8)), _full((8, IE)),
        ],
        out_specs=pl.BlockSpec((BT_BLK, IE), lambda i: (i, 0)),
        out_shape=jax.ShapeDtypeStruct((T, IE), jnp.float32),
    )(sbf, W_sbf1, W_sbf2)

    msg = xd[idx_kj] * sbf_t
    seg = jax.ops.segment_sum(msg, idx_ji, num_segments=E)

    e1, e2 = pl.pallas_call(
        _post_body,
        grid=(nb_e,),
        in_specs=[
            pl.BlockSpec((BE_BLK, IE), lambda i: (i, 0)),
            pl.BlockSpec((BE_BLK, H), lambda i: (i, 0)),
            pl.BlockSpec((BE_BLK, H), lambda i: (i, 0)),
            pl.BlockSpec((BE_BLK, 6), lambda i: (i, 0)),
            _full((IE, H)),
            _full((1, H, H)), _full((1, H)), _full((1, H, H)), _full((1, H)),
            _full((H, H)), _full((H,)),
            _full((2, H, H)), _full((2, H)), _full((2, H, H)), _full((2, H)),
            _full((6, H)),
        ],
        out_specs=[
            pl.BlockSpec((BE_BLK, H), lambda i: (i, 0)),
            pl.BlockSpec((BE_BLK, H), lambda i: (i, 0)),
        ],
        out_shape=[
            jax.ShapeDtypeStruct((E, H), jnp.float32),
            jax.ShapeDtypeStruct((E, H), jnp.float32),
        ],
    )(seg, x_ji, x1, rbf0, W_up, rb_W1, rb_b1, rb_W2, rb_b2,
      W_lin, b_lin, ra_W1, ra_b1, ra_W2, ra_b2, W_rbf)

    return (e1, e2)
